# Initial kernel scaffold; baseline (speedup 1.0000x reference)
import jax
import jax.numpy as jnp
from jax.experimental import pallas as pl


def kernel(x, spatial_types, spatial_table):
    raise NotImplementedError("write your pallas kernel here")


# speedup vs baseline: 6.4625x; 6.4625x over previous
import jax
import jax.numpy as jnp
from jax import lax
from jax.experimental import pallas as pl
from jax.experimental.pallas import tpu as pltpu
from jax.experimental.pallas import tpu_sc as plsc

_N = 50000
_B = 16
_DIM_PE = 32
_NT = 21

_L = 1568
_G = _L // 16
_STRIDE = 1563
_LAST_START = _N - _L


def _pe_body(t2_hbm, types_hbm, out_hbm, t2_v, types_v, pe_v, sem):
    cid = lax.axis_index("c")
    sid = lax.axis_index("s")
    wid = sid * 2 + cid
    start = pl.multiple_of(jnp.minimum((wid * _STRIDE) & -16, _LAST_START), 16)

    cp_t2 = pltpu.async_copy(t2_hbm, t2_v, sem)
    cp_ty = pltpu.async_copy(types_hbm.at[:, pl.ds(start, _L)], types_v, sem)
    cp_t2.wait()
    cp_ty.wait()

    viota = lax.iota(jnp.int32, 16)

    def group(g, carry):
        base16 = g * 16
        accs = [jnp.zeros((16,), jnp.float32) for _ in range(_DIM_PE)]
        for p in range(_B // 2):
            va = types_v[2 * p, pl.ds(base16, 16)]
            vb = types_v[2 * p + 1, pl.ds(base16, 16)]
            idx0 = va * (_NT * _DIM_PE) + vb * _DIM_PE
            for d in range(_DIM_PE):
                accs[d] = accs[d] + plsc.load_gather(t2_v, [idx0 + d])
        row = base16 + viota
        for d in range(_DIM_PE):
            col = jnp.full((16,), d, jnp.int32)
            plsc.store_scatter(pe_v, [row, col], accs[d])
        return carry

    lax.fori_loop(0, _G, group, 0)

    pltpu.sync_copy(pe_v, out_hbm.at[pl.ds(start, _L), :])


@jax.jit
def kernel(x, spatial_types, spatial_table):
    t2 = (spatial_table[:, None, :] + spatial_table[None, :, :]).reshape(
        _NT * _NT * _DIM_PE)

    mesh = plsc.VectorSubcoreMesh(core_axis_name="c", subcore_axis_name="s")
    pe = pl.kernel(
        _pe_body,
        out_type=jax.ShapeDtypeStruct((_N, _DIM_PE), jnp.float32),
        mesh=mesh,
        scratch_types=[
            pltpu.VMEM((_NT * _NT * _DIM_PE,), jnp.float32),
            pltpu.VMEM((_B, _L), jnp.int32),
            pltpu.VMEM((_L, _DIM_PE), jnp.float32),
            pltpu.SemaphoreType.DMA,
        ],
        compiler_params=pltpu.CompilerParams(
            use_tc_tiling_on_sc=False, needs_layout_passes=False),
        name="hetero_pe_sc",
    )(t2, spatial_types)

    return jnp.concatenate([x, pe], axis=1)


# speedup vs baseline: 13.5624x; 2.0986x over previous
import jax
import jax.numpy as jnp
from jax import lax
from jax.experimental import pallas as pl
from jax.experimental.pallas import tpu as pltpu
from jax.experimental.pallas import tpu_sc as plsc

_N = 50000
_B = 16
_DIM_PE = 32
_NT = 21

_PITCH = 33
_L = 1568
_G = _L // 16
_STRIDE = 1563
_LAST_START = _N - _L


def _pe_body(t2_hbm, types_hbm, out_hbm, t2_v, types_v, pe_v, sem):
    cid = lax.axis_index("c")
    sid = lax.axis_index("s")
    wid = sid * 2 + cid
    start = pl.multiple_of(jnp.minimum((wid * _STRIDE) & -16, _LAST_START), 16)

    cp_t2 = pltpu.async_copy(t2_hbm, t2_v, sem)
    cp_ty = pltpu.async_copy(types_hbm.at[:, pl.ds(start, _L)], types_v, sem)
    cp_t2.wait()
    cp_ty.wait()

    viota = lax.iota(jnp.int32, 16)

    def group(g, carry):
        base16 = g * 16
        accs = [jnp.zeros((16,), jnp.float32) for _ in range(_DIM_PE)]
        idx0s = []
        for p in range(_B // 2):
            va = types_v[2 * p, pl.ds(base16, 16)]
            vb = types_v[2 * p + 1, pl.ds(base16, 16)]
            idx0s.append(va * _NT + vb)
        for p in range(_B // 2):
            for d in range(_DIM_PE):
                accs[d] = accs[d] + plsc.load_gather(
                    t2_v, [idx0s[p] + d * (_NT * _NT)])
        row = base16 + viota
        for d in range(_DIM_PE):
            col = jnp.full((16,), d, jnp.int32)
            plsc.store_scatter(pe_v, [row, col], accs[d])
        return carry

    lax.fori_loop(0, _G, group, 0)

    pltpu.sync_copy(
        pe_v.at[:, pl.ds(0, _DIM_PE)], out_hbm.at[pl.ds(start, _L), :])


@jax.jit
def kernel(x, spatial_types, spatial_table):
    t2 = jnp.transpose(
        spatial_table[:, None, :] + spatial_table[None, :, :],
        (2, 0, 1)).reshape(_DIM_PE * _NT * _NT)

    mesh = plsc.VectorSubcoreMesh(core_axis_name="c", subcore_axis_name="s")
    pe = pl.kernel(
        _pe_body,
        out_type=jax.ShapeDtypeStruct((_N, _DIM_PE), jnp.float32),
        mesh=mesh,
        scratch_types=[
            pltpu.VMEM((_DIM_PE * _NT * _NT,), jnp.float32),
            pltpu.VMEM((_B, _L), jnp.int32),
            pltpu.VMEM((_L, _PITCH), jnp.float32),
            pltpu.SemaphoreType.DMA,
        ],
        compiler_params=pltpu.CompilerParams(
            use_tc_tiling_on_sc=False, needs_layout_passes=False),
        name="hetero_pe_sc",
    )(t2, spatial_types)

    return jnp.concatenate([x, pe], axis=1)
